# Initial kernel scaffold; baseline (speedup 1.0000x reference)
import jax
import jax.numpy as jnp
from jax.experimental import pallas as pl


def kernel(hidden_states, gate_w, gate_bias, w1, w2, shared_w1, shared_w2):
    raise NotImplementedError("write your pallas kernel here")


# speedup vs baseline: 1.6930x; 1.6930x over previous
import functools

import jax
import jax.numpy as jnp
from jax.experimental import pallas as pl
from jax.experimental.pallas import tpu as pltpu

TOP_K = 2
N_GROUP = 4
TOPK_GROUP = 2
ROUTED_SCALING = 2.5
NEG = -1e30


def _relu2(x):
    r = jnp.maximum(x, 0.0)
    return r * r


def _routing(x_blk, gw, gb):
    bT = x_blk.shape[0]
    E = gw.shape[0]
    gsz = E // N_GROUP
    logits = jax.lax.dot_general(
        x_blk, gw, (((1,), (1,)), ((), ())),
        preferred_element_type=jnp.float32)
    scores = jax.nn.sigmoid(logits)
    sfc = scores + gb

    def top2sum4(v):
        a, b = v[:, 0:1], v[:, 1:2]
        c, d = v[:, 2:3], v[:, 3:4]
        m_ab, n_ab = jnp.maximum(a, b), jnp.minimum(a, b)
        m_cd, n_cd = jnp.maximum(c, d), jnp.minimum(c, d)
        top1 = jnp.maximum(m_ab, m_cd)
        top2 = jnp.maximum(jnp.minimum(m_ab, m_cd), jnp.maximum(n_ab, n_cd))
        return top1 + top2

    gs = [top2sum4(sfc[:, g * gsz:(g + 1) * gsz]) for g in range(N_GROUP)]
    m_ab, n_ab = jnp.maximum(gs[0], gs[1]), jnp.minimum(gs[0], gs[1])
    m_cd, n_cd = jnp.maximum(gs[2], gs[3]), jnp.minimum(gs[2], gs[3])
    thresh = jnp.maximum(jnp.minimum(m_ab, m_cd), jnp.maximum(n_ab, n_cd))

    lane = jax.lax.broadcasted_iota(jnp.int32, (bT, E), 1)
    gid = lane // gsz
    emask = jnp.zeros((bT, E), jnp.float32)
    for g in range(N_GROUP):
        emask = emask + jnp.where(gid == g, 1.0, 0.0) * (gs[g] >= thresh)
    masked = jnp.where(emask > 0, sfc, 0.0)

    v1 = jnp.max(masked, axis=1, keepdims=True)
    idx1 = jnp.min(jnp.where(masked == v1, lane, E), axis=1, keepdims=True)
    sel1 = (lane == idx1)
    masked2 = jnp.where(sel1, NEG, masked)
    v2 = jnp.max(masked2, axis=1, keepdims=True)
    idx2 = jnp.min(jnp.where(masked2 == v2, lane, E), axis=1, keepdims=True)
    sel2 = (lane == idx2)

    w1v = jnp.sum(jnp.where(sel1, scores, 0.0), axis=1, keepdims=True)
    w2v = jnp.sum(jnp.where(sel2, scores, 0.0), axis=1, keepdims=True)
    denom = w1v + w2v + 1e-20
    combine = (ROUTED_SCALING / denom) * (
        w1v * sel1.astype(jnp.float32) + w2v * sel2.astype(jnp.float32))
    return combine


def _moe_kernel(x_ref, gw_ref, gb_ref, w1_ref, w2_ref, out_ref, comb_ref):
    e = pl.program_id(1)
    E = gw_ref.shape[0]

    @pl.when(e == 0)
    def _():
        comb_ref[...] = _routing(x_ref[...], gw_ref[...], gb_ref[...])

    lane = jax.lax.broadcasted_iota(jnp.int32, comb_ref.shape, 1)
    col = jnp.sum(jnp.where(lane == e, comb_ref[...], 0.0), axis=1,
                  keepdims=True)
    weight = jnp.where(e < E, col, 1.0)

    h = _relu2(jax.lax.dot_general(
        x_ref[...], w1_ref[0], (((1,), (1,)), ((), ())),
        preferred_element_type=jnp.float32))
    y = jax.lax.dot_general(
        h, w2_ref[0], (((1,), (1,)), ((), ())),
        preferred_element_type=jnp.float32)
    contrib = weight * y

    @pl.when(e == 0)
    def _():
        out_ref[...] = contrib

    @pl.when(e > 0)
    def _():
        out_ref[...] = out_ref[...] + contrib


@jax.jit
def kernel(hidden_states, gate_w, gate_bias, w1, w2, shared_w1, shared_w2):
    x = hidden_states
    T, D = x.shape
    E, I, _ = w1.shape
    SI = shared_w1.shape[0]
    n_shared = SI // I

    w1p = jnp.concatenate([w1, shared_w1.reshape(n_shared, I, D)], axis=0)
    w2p = jnp.concatenate(
        [w2, shared_w2.reshape(D, n_shared, I).transpose(1, 0, 2)], axis=0)
    Ep = E + n_shared
    gb2 = gate_bias.reshape(1, E)

    bT = min(1024, T)
    grid = (T // bT, Ep)

    out = pl.pallas_call(
        _moe_kernel,
        grid=grid,
        in_specs=[
            pl.BlockSpec((bT, D), lambda t, e: (t, 0)),
            pl.BlockSpec((E, D), lambda t, e: (0, 0)),
            pl.BlockSpec((1, E), lambda t, e: (0, 0)),
            pl.BlockSpec((1, I, D), lambda t, e: (e, 0, 0)),
            pl.BlockSpec((1, D, I), lambda t, e: (e, 0, 0)),
        ],
        out_specs=pl.BlockSpec((bT, D), lambda t, e: (t, 0)),
        out_shape=jax.ShapeDtypeStruct((T, D), jnp.float32),
        scratch_shapes=[pltpu.VMEM((bT, E), jnp.float32)],
    )(x, gate_w, gb2, w1p, w2p)
    return out
